# Initial kernel scaffold; baseline (speedup 1.0000x reference)
import functools

import jax
import jax.numpy as jnp
from jax import lax
from jax.experimental import pallas as pl
from jax.experimental.pallas import tpu as pltpu
from jax.experimental.pallas import tpu_sc as plsc

N = 10000
E_EDGES = 320000
D = 128
HID = 128
NEXP = 8
NLAYERS = 2
TAU = 0.8

NC = 2
NS = 16
NW = NC * NS
EPW = E_EDGES // NW
CW = 125
KCW = EPW // CW
NBUF = 4
RPT = N // NS
DEGW = 16


def _make_sc_agg(with_deg: bool):
    mesh = plsc.VectorSubcoreMesh(core_axis_name="c", subcore_axis_name="s")
    out_type = [jax.ShapeDtypeStruct((NC, N, D), jnp.float32)]
    if with_deg:
        out_type.append(jax.ShapeDtypeStruct((NC, N, DEGW), jnp.float32))

    scratch = [
        pltpu.VMEM((KCW, CW), jnp.int32),
        pltpu.VMEM((KCW, CW), jnp.int32),
    ]
    scratch += [pltpu.VMEM((CW, D), jnp.float32) for _ in range(NBUF)]
    scratch += [pltpu.VMEM_SHARED((N, D), jnp.float32)]
    if with_deg:
        scratch += [
            pltpu.VMEM_SHARED((N, DEGW), jnp.float32),
            pltpu.VMEM((CW, DEGW), jnp.float32),
        ]
    scratch += [pltpu.SemaphoreType.DMA for _ in range(NBUF)]

    def body(*refs):
        (h_hbm, src_hbm, dst_hbm, za_hbm), rest = refs[:4], refs[4:]
        if with_deg:
            (zd_hbm, on_hbm, agg_out, deg_out), rest = rest[:4], rest[4:]
        else:
            (agg_out,), rest = rest[:1], rest[1:]
        (isrc, idst), rest = rest[:2], rest[2:]
        rows_v, rest = rest[:NBUF], rest[NBUF:]
        (agg_sh,), rest = rest[:1], rest[1:]
        if with_deg:
            (deg_sh, ones_v), rest = rest[:2], rest[2:]
        sems = rest[:NBUF]

        c = lax.axis_index("c")
        s = lax.axis_index("s")
        wid = s * NC + c
        rbase = s * RPT

        pltpu.sync_copy(src_hbm.at[pl.ds(wid * KCW, KCW)], isrc)
        pltpu.sync_copy(dst_hbm.at[pl.ds(wid * KCW, KCW)], idst)
        pltpu.sync_copy(za_hbm, agg_sh.at[pl.ds(rbase, RPT)])
        if with_deg:
            pltpu.sync_copy(zd_hbm, deg_sh.at[pl.ds(rbase, RPT)])
            pltpu.sync_copy(on_hbm, ones_v)
        plsc.subcore_barrier()

        @pl.loop(0, KCW, step=NBUF)
        def _(j):
            descs = []
            for b in range(NBUF):
                descs.append(
                    pltpu.async_copy(h_hbm.at[isrc.at[j + b]], rows_v[b], sems[b])
                )
            for b in range(NBUF):
                descs[b].wait()
                pltpu.sync_copy(rows_v[b], agg_sh.at[idst.at[j + b]], add=True)
                if with_deg:
                    pltpu.sync_copy(ones_v, deg_sh.at[idst.at[j + b]], add=True)

        plsc.subcore_barrier()
        pltpu.sync_copy(agg_sh.at[pl.ds(rbase, RPT)],
                        agg_out.at[c, pl.ds(rbase, RPT)])
        if with_deg:
            pltpu.sync_copy(deg_sh.at[pl.ds(rbase, RPT)],
                            deg_out.at[c, pl.ds(rbase, RPT)])

    return pl.kernel(
        body,
        out_type=tuple(out_type),
        mesh=mesh,
        scratch_types=tuple(scratch),
        name="sc_graph_agg" + ("_deg" if with_deg else ""),
    )


_sc_agg_deg = _make_sc_agg(True)
_sc_agg = _make_sc_agg(False)


def _make_tc_dense(relu_out: bool):
    R = 1000

    def body(h_ref, ap_ref, dg_ref, wr_ref, gum_ref, w1_ref, b1_ref,
             w2_ref, b2_ref, o_ref):
        h = h_ref[...]
        a = ap_ref[0] + ap_ref[1]
        deg = jnp.maximum(dg_ref[0, :, 0:1] + dg_ref[1, :, 0:1], 1.0)
        a = a / deg
        wr = wr_ref[...]
        logits = (jnp.dot(h, wr[:D], preferred_element_type=jnp.float32)
                  + jnp.dot(a, wr[D:], preferred_element_type=jnp.float32))
        z = (logits + gum_ref[...]) / TAU
        z = z - jnp.max(z, axis=-1, keepdims=True)
        ez = jnp.exp(z)
        g = ez / jnp.sum(ez, axis=-1, keepdims=True)
        w1 = w1_ref[...]
        hexp = (jnp.dot(h, w1[:D], preferred_element_type=jnp.float32)
                + jnp.dot(a, w1[D:], preferred_element_type=jnp.float32)
                + b1_ref[...])
        hexp = jnp.maximum(hexp, 0.0)
        gm = jnp.reshape(jnp.broadcast_to(g[:, :, None], (R, NEXP, HID)),
                         (R, NEXP * HID))
        y = (jnp.dot(hexp * gm, w2_ref[...], preferred_element_type=jnp.float32)
             + jnp.dot(g, b2_ref[...], preferred_element_type=jnp.float32))
        o_ref[...] = jnp.maximum(y, 0.0) if relu_out else y

    return pl.pallas_call(
        body,
        grid=(N // R,),
        in_specs=[
            pl.BlockSpec((R, D), lambda i: (i, 0)),
            pl.BlockSpec((NC, R, D), lambda i: (0, i, 0)),
            pl.BlockSpec((NC, R, DEGW), lambda i: (0, i, 0)),
            pl.BlockSpec((2 * D, NEXP), lambda i: (0, 0)),
            pl.BlockSpec((R, NEXP), lambda i: (i, 0)),
            pl.BlockSpec((2 * D, NEXP * HID), lambda i: (0, 0)),
            pl.BlockSpec((1, NEXP * HID), lambda i: (0, 0)),
            pl.BlockSpec((NEXP * HID, D), lambda i: (0, 0)),
            pl.BlockSpec((NEXP, D), lambda i: (0, 0)),
        ],
        out_specs=pl.BlockSpec((R, D), lambda i: (i, 0)),
        out_shape=jax.ShapeDtypeStruct((N, D), jnp.float32),
        name="tc_moe_dense",
    )


_tc_dense_relu = _make_tc_dense(True)
_tc_dense_lin = _make_tc_dense(False)


@jax.jit
def _run(x, edge_index, Wr, W1, b1, W2, b2):
    src = edge_index[0].reshape(NW * KCW, CW)
    dst = edge_index[1].reshape(NW * KCW, CW)
    za = jnp.zeros((RPT, D), jnp.float32)
    zd = jnp.zeros((RPT, DEGW), jnp.float32)
    ones = jnp.ones((CW, DEGW), jnp.float32)
    gkey = jax.random.key(42)

    h = x
    degp = None
    for l in range(NLAYERS):
        u = jax.random.uniform(jax.random.fold_in(gkey, l), (N, NEXP),
                               minval=1e-6, maxval=1.0 - 1e-6)
        gum = -jnp.log(-jnp.log(u))
        if l == 0:
            aggp, degp = _sc_agg_deg(h, src, dst, za, zd, ones)
        else:
            aggp = _sc_agg(h, src, dst, za)
            if isinstance(aggp, (tuple, list)):
                aggp = aggp[0]
        w1f = jnp.transpose(W1[l], (1, 0, 2)).reshape(2 * D, NEXP * HID)
        b1f = b1[l].reshape(1, NEXP * HID)
        w2f = W2[l].reshape(NEXP * HID, D)
        tc = _tc_dense_relu if l < NLAYERS - 1 else _tc_dense_lin
        h = tc(h, aggp, degp, Wr[l], gum, w1f, b1f, w2f, b2[l])
    return h


def kernel(x, edge_index, Wr, W1, b1, W2, b2):
    return _run(x, edge_index, Wr, W1, b1, W2, b2)


# speedup vs baseline: 1.8192x; 1.8192x over previous
import functools

import jax
import jax.numpy as jnp
from jax import lax
from jax.experimental import pallas as pl
from jax.experimental.pallas import tpu as pltpu
from jax.experimental.pallas import tpu_sc as plsc

N = 10000
E_EDGES = 320000
D = 128
HID = 128
NEXP = 8
NLAYERS = 2
TAU = 0.8

NS = 16
CW = 128
KCW = 160
EPAD = NS * KCW * CW
NBUF = 2
KB = 8
NSHARD = 3
SHRP = 3456
NOUT = NSHARD * SHRP
TRASH = 8
RPT = SHRP // NS
DEGW = 16


@functools.cache
def _make_sc_agg(with_deg: bool):
    mesh = plsc.VectorSubcoreMesh(core_axis_name="c", subcore_axis_name="s",
                                  num_cores=1, num_subcores=NS)
    out_type = [jax.ShapeDtypeStruct((NOUT, D), jnp.float32)]
    if with_deg:
        out_type.append(jax.ShapeDtypeStruct((NOUT, D), jnp.float32))

    scratch = [
        pltpu.VMEM((KB, CW), jnp.int32),
        pltpu.VMEM((KB, CW), jnp.int32),
        pltpu.VMEM((KB, CW), jnp.int32),
    ]
    scratch += [pltpu.VMEM((CW, D), jnp.float32) for _ in range(NBUF)]
    scratch += [pltpu.VMEM_SHARED((SHRP + TRASH, D), jnp.float32)]
    scratch += [pltpu.SemaphoreType.DMA for _ in range(NBUF)]

    def body(*refs):
        (h_hbm, src_hbm, dst_hbm, za_hbm), rest = refs[:4], refs[4:]
        if with_deg:
            (agg_out, deg_out), rest = rest[:2], rest[2:]
        else:
            (agg_out,), rest = rest[:1], rest[1:]
        (isrc, idst, cdst), rest = rest[:3], rest[3:]
        rows_v, rest = rest[:NBUF], rest[NBUF:]
        (agg_sh,), rest = rest[:1], rest[1:]
        sems = rest[:NBUF]

        s = lax.axis_index("s")
        rbase = s * RPT

        def remap_block(blk, base, load_src):
            row0 = s * KCW + blk * KB
            if load_src:
                pltpu.sync_copy(src_hbm.at[pl.ds(row0, KB)], isrc)
            pltpu.sync_copy(dst_hbm.at[pl.ds(row0, KB)], idst)
            for k in range(KB):
                for o in range(CW // 16):
                    d = idst[k, pl.ds(o * 16, 16)]
                    ok = (d >= base) & (d < base + SHRP)
                    cdst[k, pl.ds(o * 16, 16)] = jnp.where(
                        ok, d - base, SHRP + (d & (TRASH - 1)))

        for shard in range(NSHARD):
            base = shard * SHRP

            pltpu.sync_copy(za_hbm, agg_sh.at[pl.ds(rbase, RPT)])
            plsc.subcore_barrier()

            @pl.loop(0, KCW // KB)
            def _(blk):
                remap_block(blk, base, True)
                for j in range(0, KB, NBUF):
                    descs = []
                    for b in range(NBUF):
                        descs.append(
                            pltpu.async_copy(h_hbm.at[isrc.at[j + b]],
                                             rows_v[b], sems[b])
                        )
                    for b in range(NBUF):
                        descs[b].wait()
                        pltpu.sync_copy(rows_v[b],
                                        agg_sh.at[cdst.at[j + b]], add=True)

            plsc.subcore_barrier()
            pltpu.sync_copy(agg_sh.at[pl.ds(rbase, RPT)],
                            agg_out.at[pl.ds(base + rbase, RPT)])

        if with_deg:
            @pl.loop(0, CW)
            def _(r):
                for c in range(D // 16):
                    rows_v[0][r, pl.ds(c * 16, 16)] = jnp.ones(
                        (16,), jnp.float32)

            for shard in range(NSHARD):
                base = shard * SHRP

                pltpu.sync_copy(za_hbm, agg_sh.at[pl.ds(rbase, RPT)])
                plsc.subcore_barrier()

                @pl.loop(0, KCW // KB)
                def _(blk):
                    remap_block(blk, base, False)
                    for j in range(KB):
                        pltpu.sync_copy(rows_v[0],
                                        agg_sh.at[cdst.at[j]], add=True)

                plsc.subcore_barrier()
                pltpu.sync_copy(agg_sh.at[pl.ds(rbase, RPT)],
                                deg_out.at[pl.ds(base + rbase, RPT)])

    return pl.kernel(
        body,
        out_type=tuple(out_type),
        mesh=mesh,
        scratch_types=tuple(scratch),
        name="sc_graph_agg" + ("_deg" if with_deg else ""),
    )


def _make_tc_dense(last: bool):
    R = 1000

    def body(h_ref, a_ref, dg_ref, wr_ref, gum_ref, w1_ref, b1_ref,
             w2_ref, b2_ref, o_ref):
        a = a_ref[...] / jnp.maximum(dg_ref[:, 0:1], 1.0)
        xin = jnp.concatenate([h_ref[...], a], axis=-1)
        logits = jnp.dot(xin, wr_ref[...], preferred_element_type=jnp.float32)
        z = (logits + gum_ref[...]) / TAU
        z = z - jnp.max(z, axis=-1, keepdims=True)
        ez = jnp.exp(z)
        g = ez / jnp.sum(ez, axis=-1, keepdims=True)
        hexp = jnp.dot(xin, w1_ref[...], preferred_element_type=jnp.float32) \
            + b1_ref[...]
        hexp = jnp.maximum(hexp, 0.0)
        gm = jnp.reshape(jnp.broadcast_to(g[:, :, None], (R, NEXP, HID)),
                         (R, NEXP * HID))
        y = (jnp.dot(hexp * gm, w2_ref[...], preferred_element_type=jnp.float32)
             + jnp.dot(g, b2_ref[...], preferred_element_type=jnp.float32))
        o_ref[...] = y if last else jnp.maximum(y, 0.0)

    return pl.pallas_call(
        body,
        grid=(N // R,),
        in_specs=[
            pl.BlockSpec((R, D), lambda i: (i, 0)),
            pl.BlockSpec((R, D), lambda i: (i, 0)),
            pl.BlockSpec((R, D), lambda i: (i, 0)),
            pl.BlockSpec((2 * D, NEXP), lambda i: (0, 0)),
            pl.BlockSpec((R, NEXP), lambda i: (i, 0)),
            pl.BlockSpec((2 * D, NEXP * HID), lambda i: (0, 0)),
            pl.BlockSpec((1, NEXP * HID), lambda i: (0, 0)),
            pl.BlockSpec((NEXP * HID, D), lambda i: (0, 0)),
            pl.BlockSpec((NEXP, D), lambda i: (0, 0)),
        ],
        out_specs=pl.BlockSpec((R, D), lambda i: (i, 0)),
        out_shape=jax.ShapeDtypeStruct((N, D), jnp.float32),
        name="tc_moe_dense",
    )


_tc_dense_mid = _make_tc_dense(False)
_tc_dense_last = _make_tc_dense(True)


def _tc_layer(h, agg, deg, Wr, W1, b1, W2, b2, l):
    u = jax.random.uniform(jax.random.fold_in(jax.random.key(42), l),
                           (N, NEXP), minval=1e-6, maxval=1.0 - 1e-6)
    gum = -jnp.log(-jnp.log(u))
    w1f = jnp.transpose(W1[l], (1, 0, 2)).reshape(2 * D, NEXP * HID)
    b1f = b1[l].reshape(1, NEXP * HID)
    w2f = W2[l].reshape(NEXP * HID, D)
    tc = _tc_dense_mid if l < NLAYERS - 1 else _tc_dense_last
    return tc(h, agg, deg, Wr[l], gum, w1f, b1f, w2f, b2[l])


def kernel(x, edge_index, Wr, W1, b1, W2, b2):
    npad = EPAD - E_EDGES
    pad_src = (jnp.arange(npad, dtype=jnp.int32) * 37) % N
    pad_dst = jnp.full((npad,), jnp.int32(2 ** 20))
    src = jnp.concatenate([edge_index[0], pad_src]).reshape(NS * KCW, CW)
    dst = jnp.concatenate([edge_index[1], pad_dst]).reshape(NS * KCW, CW)

    za = jnp.zeros((RPT, D), jnp.float32)

    agg, deg = _make_sc_agg(True)(x, src, dst, za)
    h = _tc_layer(x, agg, deg, Wr, W1, b1, W2, b2, 0)
    agg = _make_sc_agg(False)(h, src, dst, za)
    if isinstance(agg, (tuple, list)):
        agg = agg[0]
    return _tc_layer(h, agg, deg, Wr, W1, b1, W2, b2, 1)


# speedup vs baseline: 3.1689x; 1.7420x over previous
import functools

import jax
import jax.numpy as jnp
from jax import lax
from jax.experimental import pallas as pl
from jax.experimental.pallas import tpu as pltpu
from jax.experimental.pallas import tpu_sc as plsc

N = 10000
E_EDGES = 320000
D = 128
HID = 128
NEXP = 8
NLAYERS = 2
TAU = 0.8

NS = 16
CW = 128
KCW = 160
EPAD = NS * KCW * CW
NBUF = 2
KB = 8
NSHARD = 2
SHRP = 5248
NOUT = NSHARD * SHRP
TRASH = 8
RPT = SHRP // NS
DEGW = 16


@functools.cache
def _make_sc_agg(with_deg: bool):
    mesh = plsc.VectorSubcoreMesh(core_axis_name="c", subcore_axis_name="s",
                                  num_cores=1, num_subcores=NS)
    out_type = [jax.ShapeDtypeStruct((NOUT, D), jnp.float32)]
    if with_deg:
        out_type.append(jax.ShapeDtypeStruct((NOUT, D), jnp.float32))

    scratch = [
        pltpu.VMEM((KB, CW), jnp.int32),
        pltpu.VMEM((KB, CW), jnp.int32),
        pltpu.VMEM((KB, CW), jnp.int32),
    ]
    scratch += [pltpu.VMEM((CW, D), jnp.float32) for _ in range(NBUF)]
    scratch += [pltpu.VMEM_SHARED((SHRP + TRASH, D), jnp.float32)]
    scratch += [pltpu.SemaphoreType.DMA for _ in range(2 * NBUF)]

    def body(*refs):
        (h_hbm, src_hbm, dst_hbm, za_hbm), rest = refs[:4], refs[4:]
        if with_deg:
            (agg_out, deg_out), rest = rest[:2], rest[2:]
        else:
            (agg_out,), rest = rest[:1], rest[1:]
        (isrc, idst, cdst), rest = rest[:3], rest[3:]
        rows_v, rest = rest[:NBUF], rest[NBUF:]
        (agg_sh,), rest = rest[:1], rest[1:]
        gsems = rest[:NBUF]
        ssems = rest[NBUF:2 * NBUF]

        s = lax.axis_index("s")
        rbase = s * RPT

        def remap_block(blk, base, load_src):
            row0 = s * KCW + blk * KB
            if load_src:
                pltpu.sync_copy(src_hbm.at[pl.ds(row0, KB)], isrc)
            pltpu.sync_copy(dst_hbm.at[pl.ds(row0, KB)], idst)
            for k in range(KB):
                for o in range(CW // 16):
                    d = idst[k, pl.ds(o * 16, 16)]
                    ok = (d >= base) & (d < base + SHRP)
                    cdst[k, pl.ds(o * 16, 16)] = jnp.where(
                        ok, d - base, SHRP + (d & (TRASH - 1)))

        for shard in range(NSHARD):
            base = shard * SHRP

            pltpu.sync_copy(za_hbm, agg_sh.at[pl.ds(rbase, RPT)])
            plsc.subcore_barrier()

            @pl.loop(0, KCW // KB)
            def _(blk):
                remap_block(blk, base, True)
                gd = [None] * NBUF
                sd = [None] * NBUF
                gd[0] = pltpu.async_copy(h_hbm.at[isrc.at[0]],
                                         rows_v[0], gsems[0])
                for j in range(1, KB + 1):
                    bp = (j - 1) % NBUF
                    bc = j % NBUF
                    if j < KB:
                        if sd[bc] is not None:
                            sd[bc].wait()
                        gd[bc] = pltpu.async_copy(h_hbm.at[isrc.at[j]],
                                                  rows_v[bc], gsems[bc])
                    gd[bp].wait()
                    sd[bp] = pltpu.async_copy(rows_v[bp],
                                              agg_sh.at[cdst.at[j - 1]],
                                              ssems[bp], add=True)
                for b in range(NBUF):
                    if sd[b] is not None:
                        sd[b].wait()

            plsc.subcore_barrier()
            pltpu.sync_copy(agg_sh.at[pl.ds(rbase, RPT)],
                            agg_out.at[pl.ds(base + rbase, RPT)])

        if with_deg:
            @pl.loop(0, CW)
            def _(r):
                for c in range(D // 16):
                    rows_v[0][r, pl.ds(c * 16, 16)] = jnp.ones(
                        (16,), jnp.float32)

            for shard in range(NSHARD):
                base = shard * SHRP

                pltpu.sync_copy(za_hbm, agg_sh.at[pl.ds(rbase, RPT)])
                plsc.subcore_barrier()

                @pl.loop(0, KCW // KB)
                def _(blk):
                    remap_block(blk, base, False)
                    sd = [None] * NBUF
                    for j in range(KB):
                        b = j % NBUF
                        if sd[b] is not None:
                            sd[b].wait()
                        sd[b] = pltpu.async_copy(rows_v[0],
                                                 agg_sh.at[cdst.at[j]],
                                                 ssems[b], add=True)
                    for b in range(NBUF):
                        if sd[b] is not None:
                            sd[b].wait()

                plsc.subcore_barrier()
                pltpu.sync_copy(agg_sh.at[pl.ds(rbase, RPT)],
                                deg_out.at[pl.ds(base + rbase, RPT)])

    return pl.kernel(
        body,
        out_type=tuple(out_type),
        mesh=mesh,
        scratch_types=tuple(scratch),
        name="sc_graph_agg" + ("_deg" if with_deg else ""),
    )


def _make_tc_dense(last: bool):
    R = 1000

    def body(h_ref, a_ref, dg_ref, wr_ref, gum_ref, w1_ref, b1_ref,
             w2_ref, b2_ref, o_ref):
        a = a_ref[...] / jnp.maximum(dg_ref[:, 0:1], 1.0)
        xin = jnp.concatenate([h_ref[...], a], axis=-1)
        logits = jnp.dot(xin, wr_ref[...], preferred_element_type=jnp.float32)
        z = (logits + gum_ref[...]) / TAU
        z = z - jnp.max(z, axis=-1, keepdims=True)
        ez = jnp.exp(z)
        g = ez / jnp.sum(ez, axis=-1, keepdims=True)
        hexp = jnp.dot(xin, w1_ref[...], preferred_element_type=jnp.float32) \
            + b1_ref[...]
        hexp = jnp.maximum(hexp, 0.0)
        gm = jnp.reshape(jnp.broadcast_to(g[:, :, None], (R, NEXP, HID)),
                         (R, NEXP * HID))
        y = (jnp.dot(hexp * gm, w2_ref[...], preferred_element_type=jnp.float32)
             + jnp.dot(g, b2_ref[...], preferred_element_type=jnp.float32))
        o_ref[...] = y if last else jnp.maximum(y, 0.0)

    return pl.pallas_call(
        body,
        grid=(N // R,),
        in_specs=[
            pl.BlockSpec((R, D), lambda i: (i, 0)),
            pl.BlockSpec((R, D), lambda i: (i, 0)),
            pl.BlockSpec((R, D), lambda i: (i, 0)),
            pl.BlockSpec((2 * D, NEXP), lambda i: (0, 0)),
            pl.BlockSpec((R, NEXP), lambda i: (i, 0)),
            pl.BlockSpec((2 * D, NEXP * HID), lambda i: (0, 0)),
            pl.BlockSpec((1, NEXP * HID), lambda i: (0, 0)),
            pl.BlockSpec((NEXP * HID, D), lambda i: (0, 0)),
            pl.BlockSpec((NEXP, D), lambda i: (0, 0)),
        ],
        out_specs=pl.BlockSpec((R, D), lambda i: (i, 0)),
        out_shape=jax.ShapeDtypeStruct((N, D), jnp.float32),
        name="tc_moe_dense",
    )


_tc_dense_mid = _make_tc_dense(False)
_tc_dense_last = _make_tc_dense(True)


def _tc_layer(h, agg, deg, Wr, W1, b1, W2, b2, l):
    u = jax.random.uniform(jax.random.fold_in(jax.random.key(42), l),
                           (N, NEXP), minval=1e-6, maxval=1.0 - 1e-6)
    gum = -jnp.log(-jnp.log(u))
    w1f = jnp.transpose(W1[l], (1, 0, 2)).reshape(2 * D, NEXP * HID)
    b1f = b1[l].reshape(1, NEXP * HID)
    w2f = W2[l].reshape(NEXP * HID, D)
    tc = _tc_dense_mid if l < NLAYERS - 1 else _tc_dense_last
    return tc(h, agg, deg, Wr[l], gum, w1f, b1f, w2f, b2[l])


def kernel(x, edge_index, Wr, W1, b1, W2, b2):
    npad = EPAD - E_EDGES
    pad_src = (jnp.arange(npad, dtype=jnp.int32) * 37) % N
    pad_dst = jnp.full((npad,), jnp.int32(2 ** 20))
    src = jnp.concatenate([edge_index[0], pad_src]).reshape(NS * KCW, CW)
    dst = jnp.concatenate([edge_index[1], pad_dst]).reshape(NS * KCW, CW)

    za = jnp.zeros((RPT, D), jnp.float32)

    agg, deg = _make_sc_agg(True)(x, src, dst, za)
    h = _tc_layer(x, agg, deg, Wr, W1, b1, W2, b2, 0)
    agg = _make_sc_agg(False)(h, src, dst, za)
    if isinstance(agg, (tuple, list)):
        agg = agg[0]
    return _tc_layer(h, agg, deg, Wr, W1, b1, W2, b2, 1)


# speedup vs baseline: 3.6128x; 1.1401x over previous
import functools

import jax
import jax.numpy as jnp
from jax import lax
from jax.experimental import pallas as pl
from jax.experimental.pallas import tpu as pltpu
from jax.experimental.pallas import tpu_sc as plsc

N = 10000
E_EDGES = 320000
D = 128
HID = 128
NEXP = 8
NLAYERS = 2
TAU = 0.8

NS = 16
CW = 128
KCW = 160
EPAD = NS * KCW * CW
NBUF = 2
KB = 32
NSHARD = 2
SHRP = 5248
NOUT = NSHARD * SHRP
TRASH = 8
RPT = SHRP // NS
DEGW = 16


@functools.cache
def _make_sc_agg(with_deg: bool):
    mesh = plsc.VectorSubcoreMesh(core_axis_name="c", subcore_axis_name="s",
                                  num_cores=1, num_subcores=NS)
    out_type = [jax.ShapeDtypeStruct((NOUT, D), jnp.float32)]
    if with_deg:
        out_type.append(jax.ShapeDtypeStruct((NOUT, D), jnp.float32))

    scratch = [
        pltpu.VMEM((KB, CW), jnp.int32),
        pltpu.VMEM((KB, CW), jnp.int32),
        pltpu.VMEM((KB, CW), jnp.int32),
    ]
    scratch += [pltpu.VMEM((CW, D), jnp.float32) for _ in range(NBUF)]
    scratch += [pltpu.VMEM_SHARED((SHRP + TRASH, D), jnp.float32)]
    scratch += [pltpu.SemaphoreType.DMA for _ in range(2 * NBUF)]

    def body(*refs):
        (h_hbm, src_hbm, dst_hbm, za_hbm), rest = refs[:4], refs[4:]
        if with_deg:
            (agg_out, deg_out), rest = rest[:2], rest[2:]
        else:
            (agg_out,), rest = rest[:1], rest[1:]
        (isrc, idst, cdst), rest = rest[:3], rest[3:]
        rows_v, rest = rest[:NBUF], rest[NBUF:]
        (agg_sh,), rest = rest[:1], rest[1:]
        gsems = rest[:NBUF]
        ssems = rest[NBUF:2 * NBUF]

        s = lax.axis_index("s")
        rbase = s * RPT

        def remap_block(blk, base, load_src):
            row0 = s * KCW + blk * KB
            if load_src:
                pltpu.sync_copy(src_hbm.at[pl.ds(row0, KB)], isrc)
            pltpu.sync_copy(dst_hbm.at[pl.ds(row0, KB)], idst)
            for k in range(KB):
                for o in range(CW // 16):
                    d = idst[k, pl.ds(o * 16, 16)]
                    ok = (d >= base) & (d < base + SHRP)
                    cdst[k, pl.ds(o * 16, 16)] = jnp.where(
                        ok, d - base, SHRP + (d & (TRASH - 1)))

        for shard in range(NSHARD):
            base = shard * SHRP

            pltpu.sync_copy(za_hbm, agg_sh.at[pl.ds(rbase, RPT)])
            plsc.subcore_barrier()

            @pl.loop(0, KCW // KB)
            def _(blk):
                remap_block(blk, base, True)
                gd = [None] * NBUF
                sd = [None] * NBUF
                gd[0] = pltpu.async_copy(h_hbm.at[isrc.at[0]],
                                         rows_v[0], gsems[0])
                for j in range(1, KB + 1):
                    bp = (j - 1) % NBUF
                    bc = j % NBUF
                    if j < KB:
                        if sd[bc] is not None:
                            sd[bc].wait()
                        gd[bc] = pltpu.async_copy(h_hbm.at[isrc.at[j]],
                                                  rows_v[bc], gsems[bc])
                    gd[bp].wait()
                    sd[bp] = pltpu.async_copy(rows_v[bp],
                                              agg_sh.at[cdst.at[j - 1]],
                                              ssems[bp], add=True)
                for b in range(NBUF):
                    if sd[b] is not None:
                        sd[b].wait()

            plsc.subcore_barrier()
            pltpu.sync_copy(agg_sh.at[pl.ds(rbase, RPT)],
                            agg_out.at[pl.ds(base + rbase, RPT)])

        if with_deg:
            @pl.loop(0, CW)
            def _(r):
                for c in range(D // 16):
                    rows_v[0][r, pl.ds(c * 16, 16)] = jnp.ones(
                        (16,), jnp.float32)

            for shard in range(NSHARD):
                base = shard * SHRP

                pltpu.sync_copy(za_hbm, agg_sh.at[pl.ds(rbase, RPT)])
                plsc.subcore_barrier()

                @pl.loop(0, KCW // KB)
                def _(blk):
                    remap_block(blk, base, False)
                    sd = [None] * NBUF
                    for j in range(KB):
                        b = j % NBUF
                        if sd[b] is not None:
                            sd[b].wait()
                        sd[b] = pltpu.async_copy(rows_v[0],
                                                 agg_sh.at[cdst.at[j]],
                                                 ssems[b], add=True)
                    for b in range(NBUF):
                        if sd[b] is not None:
                            sd[b].wait()

                plsc.subcore_barrier()
                pltpu.sync_copy(agg_sh.at[pl.ds(rbase, RPT)],
                                deg_out.at[pl.ds(base + rbase, RPT)])

    return pl.kernel(
        body,
        out_type=tuple(out_type),
        mesh=mesh,
        scratch_types=tuple(scratch),
        name="sc_graph_agg" + ("_deg" if with_deg else ""),
    )


def _make_tc_dense(last: bool):
    R = 1000

    def body(h_ref, a_ref, dg_ref, wr_ref, gum_ref, w1_ref, b1_ref,
             w2_ref, b2_ref, o_ref):
        a = a_ref[...] / jnp.maximum(dg_ref[:, 0:1], 1.0)
        xin = jnp.concatenate([h_ref[...], a], axis=-1)
        logits = jnp.dot(xin, wr_ref[...], preferred_element_type=jnp.float32)
        z = (logits + gum_ref[...]) / TAU
        z = z - jnp.max(z, axis=-1, keepdims=True)
        ez = jnp.exp(z)
        g = ez / jnp.sum(ez, axis=-1, keepdims=True)
        hexp = jnp.dot(xin, w1_ref[...], preferred_element_type=jnp.float32) \
            + b1_ref[...]
        hexp = jnp.maximum(hexp, 0.0)
        gm = jnp.reshape(jnp.broadcast_to(g[:, :, None], (R, NEXP, HID)),
                         (R, NEXP * HID))
        y = (jnp.dot(hexp * gm, w2_ref[...], preferred_element_type=jnp.float32)
             + jnp.dot(g, b2_ref[...], preferred_element_type=jnp.float32))
        o_ref[...] = y if last else jnp.maximum(y, 0.0)

    return pl.pallas_call(
        body,
        grid=(N // R,),
        in_specs=[
            pl.BlockSpec((R, D), lambda i: (i, 0)),
            pl.BlockSpec((R, D), lambda i: (i, 0)),
            pl.BlockSpec((R, D), lambda i: (i, 0)),
            pl.BlockSpec((2 * D, NEXP), lambda i: (0, 0)),
            pl.BlockSpec((R, NEXP), lambda i: (i, 0)),
            pl.BlockSpec((2 * D, NEXP * HID), lambda i: (0, 0)),
            pl.BlockSpec((1, NEXP * HID), lambda i: (0, 0)),
            pl.BlockSpec((NEXP * HID, D), lambda i: (0, 0)),
            pl.BlockSpec((NEXP, D), lambda i: (0, 0)),
        ],
        out_specs=pl.BlockSpec((R, D), lambda i: (i, 0)),
        out_shape=jax.ShapeDtypeStruct((N, D), jnp.float32),
        name="tc_moe_dense",
    )


_tc_dense_mid = _make_tc_dense(False)
_tc_dense_last = _make_tc_dense(True)


def _tc_layer(h, agg, deg, Wr, W1, b1, W2, b2, l):
    u = jax.random.uniform(jax.random.fold_in(jax.random.key(42), l),
                           (N, NEXP), minval=1e-6, maxval=1.0 - 1e-6)
    gum = -jnp.log(-jnp.log(u))
    w1f = jnp.transpose(W1[l], (1, 0, 2)).reshape(2 * D, NEXP * HID)
    b1f = b1[l].reshape(1, NEXP * HID)
    w2f = W2[l].reshape(NEXP * HID, D)
    tc = _tc_dense_mid if l < NLAYERS - 1 else _tc_dense_last
    return tc(h, agg, deg, Wr[l], gum, w1f, b1f, w2f, b2[l])


def kernel(x, edge_index, Wr, W1, b1, W2, b2):
    npad = EPAD - E_EDGES
    pad_src = (jnp.arange(npad, dtype=jnp.int32) * 37) % N
    pad_dst = jnp.full((npad,), jnp.int32(2 ** 20))
    src = jnp.concatenate([edge_index[0], pad_src]).reshape(NS * KCW, CW)
    dst = jnp.concatenate([edge_index[1], pad_dst]).reshape(NS * KCW, CW)

    za = jnp.zeros((RPT, D), jnp.float32)

    agg, deg = _make_sc_agg(True)(x, src, dst, za)
    h = _tc_layer(x, agg, deg, Wr, W1, b1, W2, b2, 0)
    agg = _make_sc_agg(False)(h, src, dst, za)
    if isinstance(agg, (tuple, list)):
        agg = agg[0]
    return _tc_layer(h, agg, deg, Wr, W1, b1, W2, b2, 1)
